# baseline (device time: 101537 ns/iter reference)
import jax
import jax.numpy as jnp
from jax import lax
from jax.experimental import pallas as pl
from jax.experimental.pallas import tpu as pltpu


def kernel(x, pi):
    shard_shape = x.shape

    def body(x_ref, pi_ref, out_ref, vmem_ref, send_sem, recv_sem, copy_sem):
        my_x = lax.axis_index("x")
        my_y = lax.axis_index("y")
        dst_y = jnp.where(my_y == 0, pi_ref[0], pi_ref[1])
        barrier_sem = pltpu.get_barrier_semaphore()

        @pl.when(dst_y == my_y)
        def _local():
            cp = pltpu.make_async_copy(x_ref, out_ref, copy_sem)
            cp.start()
            cp.wait()

        @pl.when(dst_y != my_y)
        def _swap():
            stage = pltpu.make_async_copy(x_ref, vmem_ref, copy_sem)
            stage.start()
            pl.semaphore_signal(
                barrier_sem,
                inc=1,
                device_id=(my_x, dst_y),
                device_id_type=pl.DeviceIdType.MESH,
            )
            pl.semaphore_wait(barrier_sem, 1)
            stage.wait()
            rdma = pltpu.make_async_remote_copy(
                src_ref=vmem_ref,
                dst_ref=out_ref,
                send_sem=send_sem,
                recv_sem=recv_sem,
                device_id=(my_x, dst_y),
                device_id_type=pl.DeviceIdType.MESH,
            )
            rdma.start()
            rdma.wait()

    return pl.pallas_call(
        body,
        out_shape=jax.ShapeDtypeStruct(shard_shape, x.dtype),
        in_specs=[
            pl.BlockSpec(memory_space=pl.ANY),
            pl.BlockSpec(memory_space=pltpu.SMEM),
        ],
        out_specs=pl.BlockSpec(memory_space=pl.ANY),
        scratch_shapes=[
            pltpu.VMEM(shard_shape, x.dtype),
            pltpu.SemaphoreType.DMA,
            pltpu.SemaphoreType.DMA,
            pltpu.SemaphoreType.DMA,
        ],
        compiler_params=pltpu.CompilerParams(collective_id=0),
    )(x, pi)


# device time: 100952 ns/iter; 1.0058x vs baseline; 1.0058x over previous
import jax
import jax.numpy as jnp
from jax import lax
from jax.experimental import pallas as pl
from jax.experimental.pallas import tpu as pltpu


def kernel(x, pi):
    shard_shape = x.shape

    def body(x_ref, pi_ref, out_ref, send_sem, recv_sem, copy_sem):
        my_x = lax.axis_index("x")
        my_y = lax.axis_index("y")
        dst_y = jnp.where(my_y == 0, pi_ref[0], pi_ref[1])
        barrier_sem = pltpu.get_barrier_semaphore()

        @pl.when(dst_y == my_y)
        def _local():
            cp = pltpu.make_async_copy(x_ref, out_ref, copy_sem)
            cp.start()
            cp.wait()

        @pl.when(dst_y != my_y)
        def _swap():
            pl.semaphore_signal(
                barrier_sem,
                inc=1,
                device_id=(my_x, dst_y),
                device_id_type=pl.DeviceIdType.MESH,
            )
            pl.semaphore_wait(barrier_sem, 1)
            rdma = pltpu.make_async_remote_copy(
                src_ref=x_ref,
                dst_ref=out_ref,
                send_sem=send_sem,
                recv_sem=recv_sem,
                device_id=(my_x, dst_y),
                device_id_type=pl.DeviceIdType.MESH,
            )
            rdma.start()
            rdma.wait()

    return pl.pallas_call(
        body,
        out_shape=jax.ShapeDtypeStruct(shard_shape, x.dtype),
        in_specs=[
            pl.BlockSpec(memory_space=pl.ANY),
            pl.BlockSpec(memory_space=pltpu.SMEM),
        ],
        out_specs=pl.BlockSpec(memory_space=pl.ANY),
        scratch_shapes=[
            pltpu.SemaphoreType.DMA,
            pltpu.SemaphoreType.DMA,
            pltpu.SemaphoreType.DMA,
        ],
        compiler_params=pltpu.CompilerParams(collective_id=0),
    )(x, pi)
